# Initial kernel scaffold; baseline (speedup 1.0000x reference)
import jax
import jax.numpy as jnp
from jax.experimental import pallas as pl


def kernel(x, edge_index, W1, b1, g1, be1, W2, b2, g2, be2, W3, b3):
    raise NotImplementedError("write your pallas kernel here")


# speedup vs baseline: 153.7171x; 153.7171x over previous
import functools

import jax
import jax.numpy as jnp
import numpy as np
from jax import lax
from jax.experimental import pallas as pl
from jax.experimental.pallas import tpu as pltpu
from jax.experimental.pallas import tpu_sc as plsc

N = 10000
E = 320000
D_IN = 128
D_HID = 128
D_OUT = 40
D_OUT_PAD = 128
BN_EPS = 1e-5

NC = 2
NS = 16
NW = NC * NS
K = 128
C = -(-E // (NW * K))
E_PAD = NW * C * K
N_ACC = N + 112
RP_ACC = N_ACC // NS

ROWS_BLK = 1024


def _sc_mesh():
    return plsc.VectorSubcoreMesh(
        core_axis_name="c", subcore_axis_name="s", num_cores=NC, num_subcores=NS
    )


def _sc_degree(col3, ones_rows, zeros128):
    @functools.partial(
        pl.kernel,
        out_type=jax.ShapeDtypeStruct((NC, N_ACC, 128), jnp.float32),
        mesh=_sc_mesh(),
        scratch_types=[
            pltpu.VMEM((C, K), jnp.int32),
            pltpu.VMEM((K, 128), jnp.float32),
            pltpu.VMEM_SHARED((N_ACC, 128), jnp.float32),
        ],
    )
    def k(col_hbm, ones_hbm, zeros_hbm, out_hbm, colb, ones_v, acc):
        c = lax.axis_index("c")
        s = lax.axis_index("s")
        w = c * NS + s
        pltpu.sync_copy(zeros_hbm.at[pl.ds(s * RP_ACC, RP_ACC)],
                        acc.at[pl.ds(s * RP_ACC, RP_ACC)])
        pltpu.sync_copy(col_hbm.at[w], colb)
        pltpu.sync_copy(ones_hbm, ones_v)
        plsc.subcore_barrier()

        def body(g, carry):
            pltpu.sync_copy(ones_v, acc.at[colb.at[g]], add=True)
            return carry

        lax.fori_loop(jnp.int32(0), jnp.int32(C), body, jnp.int32(0))
        plsc.subcore_barrier()
        pltpu.sync_copy(acc.at[pl.ds(s * RP_ACC, RP_ACC)],
                        out_hbm.at[c, pl.ds(s * RP_ACC, RP_ACC)])

    return k(col3, ones_rows, zeros128)


def _sc_scatter(hp, row3, col3, zeros_acc, d):
    @functools.partial(
        pl.kernel,
        out_type=jax.ShapeDtypeStruct((NC, N_ACC, d), jnp.float32),
        mesh=_sc_mesh(),
        scratch_types=[
            pltpu.VMEM((C, K), jnp.int32),
            pltpu.VMEM((C, K), jnp.int32),
            pltpu.VMEM((K, d), jnp.float32),
            pltpu.VMEM_SHARED((N_ACC, d), jnp.float32),
            pltpu.SemaphoreType.DMA,
        ],
    )
    def k(hp_hbm, row_hbm, col_hbm, zeros_hbm, out_hbm, rowb, colb, rows, acc, sem):
        c = lax.axis_index("c")
        s = lax.axis_index("s")
        w = c * NS + s
        pltpu.sync_copy(zeros_hbm.at[pl.ds(s * RP_ACC, RP_ACC)],
                        acc.at[pl.ds(s * RP_ACC, RP_ACC)])
        pltpu.sync_copy(row_hbm.at[w], rowb)
        pltpu.sync_copy(col_hbm.at[w], colb)
        plsc.subcore_barrier()

        def body(g, carry):
            pltpu.async_copy(hp_hbm.at[rowb.at[g]], rows, sem).wait()
            pltpu.sync_copy(rows, acc.at[colb.at[g]], add=True)
            return carry

        lax.fori_loop(jnp.int32(0), jnp.int32(C), body, jnp.int32(0))
        plsc.subcore_barrier()
        pltpu.sync_copy(acc.at[pl.ds(s * RP_ACC, RP_ACC)],
                        out_hbm.at[c, pl.ds(s * RP_ACC, RP_ACC)])

    return k(hp, row3, col3, zeros_acc)


def _tc_layer1(x, W1, hist):
    grid = (pl.cdiv(N, ROWS_BLK),)

    def body(x_ref, w_ref, h_ref, hp_ref, dv_ref):
        deg = 1.0 + h_ref[0][:, 0:1] + h_ref[1][:, 0:1]
        dinv = lax.rsqrt(deg)
        h = jnp.dot(x_ref[...], w_ref[...], preferred_element_type=jnp.float32)
        hp_ref[...] = h * dinv
        dv_ref[...] = jnp.broadcast_to(dinv, (ROWS_BLK, 16))

    return pl.pallas_call(
        body,
        grid=grid,
        in_specs=[
            pl.BlockSpec((ROWS_BLK, D_IN), lambda i: (i, np.int32(0))),
            pl.BlockSpec((D_IN, D_HID), lambda i: (np.int32(0), np.int32(0))),
            pl.BlockSpec((NC, ROWS_BLK, 128), lambda i: (np.int32(0), i, np.int32(0))),
        ],
        out_specs=[
            pl.BlockSpec((ROWS_BLK, D_HID), lambda i: (i, np.int32(0))),
            pl.BlockSpec((ROWS_BLK, 16), lambda i: (i, np.int32(0))),
        ],
        out_shape=[
            jax.ShapeDtypeStruct((N, D_HID), jnp.float32),
            jax.ShapeDtypeStruct((N, 16), jnp.float32),
        ],
    )(x, W1, hist)


def _tc_mid(S, hp, dv, b, gs, be, W, d_out):
    grid = (pl.cdiv(N, ROWS_BLK),)
    d_in = hp.shape[1]

    def body(s_ref, hp_ref, dv_ref, b_ref, gs_ref, be_ref, w_ref, out_ref):
        dinv = dv_ref[:, 0:1]
        y = dinv * (s_ref[0] + s_ref[1] + hp_ref[...]) + b_ref[...]
        y = y * gs_ref[...] + be_ref[...]
        y = jnp.maximum(y, 0.0)
        out_ref[...] = jnp.dot(y, w_ref[...], preferred_element_type=jnp.float32) * dinv

    return pl.pallas_call(
        body,
        grid=grid,
        in_specs=[
            pl.BlockSpec((NC, ROWS_BLK, d_in), lambda i: (np.int32(0), i, np.int32(0))),
            pl.BlockSpec((ROWS_BLK, d_in), lambda i: (i, np.int32(0))),
            pl.BlockSpec((ROWS_BLK, 16), lambda i: (i, np.int32(0))),
            pl.BlockSpec((1, d_in), lambda i: (np.int32(0), np.int32(0))),
            pl.BlockSpec((1, d_in), lambda i: (np.int32(0), np.int32(0))),
            pl.BlockSpec((1, d_in), lambda i: (np.int32(0), np.int32(0))),
            pl.BlockSpec((d_in, d_out), lambda i: (np.int32(0), np.int32(0))),
        ],
        out_specs=pl.BlockSpec((ROWS_BLK, d_out), lambda i: (i, np.int32(0))),
        out_shape=jax.ShapeDtypeStruct((N, d_out), jnp.float32),
    )(S, hp, dv, b, gs, be, W)


def _tc_final(S, hp, dv, b3):
    grid = (pl.cdiv(N, ROWS_BLK),)

    def body(s_ref, hp_ref, dv_ref, b_ref, out_ref):
        dinv = dv_ref[:, 0:1]
        y = dinv * (s_ref[0] + s_ref[1] + hp_ref[...]) + b_ref[...]
        mask = lax.broadcasted_iota(jnp.int32, (1, D_OUT_PAD), 1) < D_OUT
        neg = jnp.float32(-jnp.inf)
        ylog = jnp.where(mask, y, neg)
        m = jnp.max(ylog, axis=1, keepdims=True)
        e = jnp.where(mask, jnp.exp(ylog - m), 0.0)
        p = e / jnp.sum(e, axis=1, keepdims=True)
        out_ref[...] = p[:, :D_OUT]

    return pl.pallas_call(
        body,
        grid=grid,
        in_specs=[
            pl.BlockSpec((NC, ROWS_BLK, D_OUT_PAD), lambda i: (np.int32(0), i, np.int32(0))),
            pl.BlockSpec((ROWS_BLK, D_OUT_PAD), lambda i: (i, np.int32(0))),
            pl.BlockSpec((ROWS_BLK, 16), lambda i: (i, np.int32(0))),
            pl.BlockSpec((1, D_OUT_PAD), lambda i: (np.int32(0), np.int32(0))),
        ],
        out_specs=pl.BlockSpec((ROWS_BLK, D_OUT), lambda i: (i, np.int32(0))),
        out_shape=jax.ShapeDtypeStruct((N, D_OUT), jnp.float32),
    )(S, hp, dv, b3)


def kernel(x, edge_index, W1, b1, g1, be1, W2, b2, g2, be2, W3, b3):
    f32 = jnp.float32
    x = x.astype(f32)
    W1 = W1.astype(f32)
    W2 = W2.astype(f32)
    W3 = W3.astype(f32)
    row = edge_index[0].astype(jnp.int32)
    col = edge_index[1].astype(jnp.int32)
    pad = E_PAD - E
    row3 = jnp.concatenate([row, jnp.zeros((pad,), jnp.int32)]).reshape(NW, C, K)
    col3 = jnp.concatenate([col, jnp.full((pad,), N, jnp.int32)]).reshape(NW, C, K)

    zeros128 = jnp.zeros((N_ACC, D_HID), f32)

    bn_scale = np.float32(1.0 / np.sqrt(1.0 + BN_EPS))
    b1r = b1.reshape(1, D_HID).astype(f32)
    gs1 = (g1 * bn_scale).reshape(1, D_HID).astype(f32)
    be1r = be1.reshape(1, D_HID).astype(f32)
    b2r = b2.reshape(1, D_HID).astype(f32)
    gs2 = (g2 * bn_scale).reshape(1, D_HID).astype(f32)
    be2r = be2.reshape(1, D_HID).astype(f32)
    W3p = jnp.pad(W3, ((0, 0), (0, D_OUT_PAD - D_OUT))).astype(f32)
    b3r = jnp.pad(b3, (0, D_OUT_PAD - D_OUT)).reshape(1, D_OUT_PAD).astype(f32)

    ones_rows = jnp.ones((K, 128), f32)
    hist = _sc_degree(col3, ones_rows, zeros128)
    h1p, dv = _tc_layer1(x, W1, hist)
    S1 = _sc_scatter(h1p, row3, col3, zeros128, D_HID)
    h2p = _tc_mid(S1, h1p, dv, b1r, gs1, be1r, W2, D_HID)
    S2 = _sc_scatter(h2p, row3, col3, zeros128, D_HID)
    h3p = _tc_mid(S2, h2p, dv, b2r, gs2, be2r, W3p, D_OUT_PAD)
    S3 = _sc_scatter(h3p, row3, col3, zeros128, D_OUT_PAD)
    return _tc_final(S3, h3p, dv, b3r).astype(jnp.float64)
